# baseline (speedup 1.0000x reference)
import functools

import jax
import jax.numpy as jnp
from jax import lax
from jax.experimental import pallas as pl
from jax.experimental.pallas import tpu as pltpu
from jax.experimental.pallas import tpu_sc as plsc

_N = 10000
_NP = 10240
_NH = _NP // 2
_E = 160000
_EP = 163840
_D = 256
_H = 128
_NS = 16
_CH = 128
_EC = 64
_NCH = _EP // _NS // _EC
_RPH = _NH // _NS

_mesh = plsc.VectorSubcoreMesh(core_axis_name="c", subcore_axis_name="s")


@functools.partial(
    pl.kernel,
    out_type=jax.ShapeDtypeStruct((_NP // _CH, _CH), jnp.float32),
    mesh=_mesh,
    scratch_types=[
        pltpu.VMEM((_EP // _NS,), jnp.int32),
        pltpu.VMEM((_EP // _NS,), jnp.float32),
        pltpu.VMEM((_NP // _CH, _CH), jnp.float32),
        pltpu.VMEM((_NP // _CH,), jnp.int32),
        pltpu.VMEM_SHARED((_NP // _CH, _CH), jnp.float32),
    ],
    compiler_params=pltpu.CompilerParams(needs_layout_passes=False),
)
def _deg_kernel(dst_hbm, val_hbm, deg_hbm, dstv, valv, part, idv, shdeg):
    c = lax.axis_index("c")
    s = lax.axis_index("s")

    @pl.when(c == 0)
    def _core0():
        nrow = _NP // _CH
        def zrow(i, carry):
            for k in range(_CH // 16):
                part[i, pl.ds(k * 16, 16)] = jnp.zeros((16,), jnp.float32)
            return carry
        lax.fori_loop(0, nrow, zrow, 0)
        pltpu.sync_copy(part.at[pl.ds(0, nrow // _NS)],
                        shdeg.at[pl.ds(s * (nrow // _NS), nrow // _NS)])
        for k in range(nrow // 16):
            idv[pl.ds(k * 16, 16)] = lax.iota(jnp.int32, 16) + (k * 16)
        npe = _EP // _NS
        pltpu.sync_copy(dst_hbm.at[pl.ds(s * npe, npe)], dstv)
        pltpu.sync_copy(val_hbm.at[pl.ds(s * npe, npe)], valv)
        plsc.subcore_barrier()
        def sbody(i, carry):
            d16 = dstv[pl.ds(i * 16, 16)]
            v16 = valv[pl.ds(i * 16, 16)]
            r16 = lax.shift_right_logical(d16, 7)
            c16 = lax.bitwise_and(d16, 127)
            plsc.addupdate_scatter(part, [r16, c16], v16)
            return carry
        lax.fori_loop(0, npe // 16, sbody, 0)
        pltpu.sync_copy(part, shdeg.at[idv], add=True)
        plsc.subcore_barrier()

        @pl.when(s == 0)
        def _writer():
            pltpu.sync_copy(shdeg, part)
            pltpu.sync_copy(part, deg_hbm)


@functools.partial(
    pl.kernel,
    out_type=jax.ShapeDtypeStruct((2, _NH, 2, _H), jnp.float32),
    mesh=_mesh,
    scratch_types=[
        pltpu.VMEM((2, _EC), jnp.int32),
        pltpu.VMEM((2, _EC), jnp.int32),
        pltpu.VMEM((_EC, 2, _H), jnp.float32),
        pltpu.VMEM((_EC, 2, _H), jnp.float32),
        pltpu.VMEM_SHARED((_NH, 2, _H), jnp.float32),
        pltpu.VMEM((16,), jnp.int32),
        pltpu.SemaphoreType.DMA,
        pltpu.SemaphoreType.DMA,
    ],
    compiler_params=pltpu.CompilerParams(needs_layout_passes=False),
)
def _msg_kernel(g_hbm, idx_hbm, trips_hbm, out_hbm, ib0, ib1, rows0, rows1,
                accum, tsm, sem0, sem1):
    c = lax.axis_index("c")
    s = lax.axis_index("s")
    pltpu.sync_copy(trips_hbm, tsm)
    def zrow(i, carry):
        for q in range(2):
            for k in range(_H // 16):
                rows0[i, q, pl.ds(k * 16, 16)] = jnp.zeros((16,), jnp.float32)
        return carry
    lax.fori_loop(0, _EC, zrow, 0)
    base = s * _RPH
    for k in range(_RPH // _EC):
        pltpu.sync_copy(rows0, accum.at[pl.ds(base + k * _EC, _EC)])
    plsc.subcore_barrier()

    tv = tsm[pl.ds(0, 16)]
    ntrip = jnp.sum(jnp.where(lax.iota(jnp.int32, 16) == c, tv, 0))
    nch = ntrip * 2
    it = idx_hbm.at[c, s]
    pltpu.sync_copy(it.at[0], ib0)
    pltpu.async_copy(g_hbm.at[ib0.at[0]], rows0, sem0)
    pltpu.sync_copy(it.at[1], ib1)
    pltpu.async_copy(g_hbm.at[ib1.at[0]], rows1, sem1)
    bufs = ((ib0, rows0, sem0), (ib1, rows1, sem1))

    def step(i, carry):
        jb = i * 2
        for b in range(2):
            j = jb + b
            ib, rows, sem = bufs[b]
            pltpu.make_async_copy(g_hbm.at[ib.at[0]], rows, sem).wait()
            pltpu.sync_copy(rows, accum.at[ib.at[1]], add=True)

            @pl.when(j + 2 < nch)
            def _prefetch():
                pltpu.sync_copy(it.at[j + 2], ib)
                pltpu.async_copy(g_hbm.at[ib.at[0]], rows, sem)
        return carry

    lax.fori_loop(0, ntrip, step, 0)
    plsc.subcore_barrier()
    pltpu.sync_copy(accum.at[pl.ds(base, _RPH)],
                    out_hbm.at[c].at[pl.ds(base, _RPH)])


def _init_body(feat_ref, win_ref, bin_ref, w1_ref, deg_ref,
               h_ref, g_ref, dinv_ref):
    j = pl.program_id(0)
    hn = jnp.dot(feat_ref[...], win_ref[...],
                 preferred_element_type=jnp.float32) + bin_ref[...]
    rid = j * _D + lax.broadcasted_iota(jnp.int32, (_D, 1), 0)
    dinv = jnp.where(rid < _N, lax.rsqrt(deg_ref[...] + 1.0), 0.0)
    g = dinv * jnp.dot(hn, w1_ref[...], preferred_element_type=jnp.float32)
    h_ref[...] = hn
    g_ref[...] = g
    dinv_ref[...] = dinv


def _layer_body(h_ref, a_ref, g_ref, dinv_ref, b_ref, w_ref, ho_ref, go_ref):
    dinv = dinv_ref[...]
    hn = jnp.maximum(h_ref[...] + dinv * (a_ref[...] + g_ref[...])
                     + b_ref[...], 0.0)
    gn = dinv * jnp.dot(hn, w_ref[...], preferred_element_type=jnp.float32)
    ho_ref[...] = hn
    go_ref[...] = gn


def _head_body(h_ref, a_ref, g_ref, dinv_ref, b_ref, wo1_ref, bo1_ref,
               wo2_ref, o_ref):
    dinv = dinv_ref[...]
    hn = jnp.maximum(h_ref[...] + dinv * (a_ref[...] + g_ref[...])
                     + b_ref[...], 0.0)
    t = jnp.dot(hn, wo1_ref[...], preferred_element_type=jnp.float32)
    t = t + bo1_ref[...]
    t = jnp.where(t >= 0, t, 0.01 * t)
    y = jnp.dot(t, wo2_ref[...], preferred_element_type=jnp.float32)
    o_ref[...] = y


_GRID = _NP // _D


def _full(shape):
    return pl.BlockSpec(shape, lambda j: tuple(0 for _ in shape))


_ROW = pl.BlockSpec((_D, _D), lambda j: (j, 0))
_COL = pl.BlockSpec((_D, 1), lambda j: (j, 0))


def _tc_init(feat, w_in, b_in, w1, deg):
    return pl.pallas_call(
        _init_body,
        grid=(_GRID,),
        in_specs=[
            pl.BlockSpec((_D, 8), lambda j: (j, 0)),
            _full((8, _D)),
            _full((1, _D)),
            _full((_D, _D)),
            _COL,
        ],
        out_specs=[_ROW, _ROW, _COL],
        out_shape=[
            jax.ShapeDtypeStruct((_NP, _D), jnp.float32),
            jax.ShapeDtypeStruct((_NP, _D), jnp.float32),
            jax.ShapeDtypeStruct((_NP, 1), jnp.float32),
        ],
    )(feat, w_in, b_in, w1, deg)


def _tc_layer(h, acc, g, dinv, b, w_next):
    return pl.pallas_call(
        _layer_body,
        grid=(_GRID,),
        in_specs=[_ROW, _ROW, _ROW, _COL, _full((1, _D)), _full((_D, _D))],
        out_specs=[_ROW, _ROW],
        out_shape=[
            jax.ShapeDtypeStruct((_NP, _D), jnp.float32),
            jax.ShapeDtypeStruct((_NP, _D), jnp.float32),
        ],
    )(h, acc, g, dinv, b, w_next)


def _tc_head(h, acc, g, dinv, b, w_o1, b_o1, w_o2):
    return pl.pallas_call(
        _head_body,
        grid=(_GRID,),
        in_specs=[_ROW, _ROW, _ROW, _COL, _full((1, _D)), _full((_D, _D)),
                  _full((1, _D)), _full((_D, 1))],
        out_specs=_COL,
        out_shape=jax.ShapeDtypeStruct((_NP, 1), jnp.float32),
    )(h, acc, g, dinv, b, w_o1, b_o1, w_o2)


def kernel(x, pos, edge_index, W_in, b_in, W1, b1, W2, b2, W3, b3, W4, b4,
           W_o1, b_o1, W_o2, b_o2):
    f32 = jnp.float32
    i32 = jnp.int32
    src = edge_index[0]
    dst = edge_index[1]
    pad = _EP - _E

    dst_p = jnp.concatenate([dst, jnp.full((pad,), _N, i32)])
    val_p = jnp.concatenate([jnp.ones((_E,), f32), jnp.zeros((pad,), f32)])

    m1 = dst >= _NH
    c1 = jnp.cumsum(m1.astype(i32))
    n1 = c1[-1]
    n0 = _E - n1
    c0 = jnp.cumsum((~m1).astype(i32))
    pos_ = jnp.where(m1, n0 + c1 - 1, c0 - 1)
    ssrc = jnp.zeros((_EP,), i32).at[pos_].set(src)
    sdst = jnp.zeros((_EP,), i32).at[pos_].set(dst)

    ar = jnp.arange(_EP, dtype=i32)
    src0 = jnp.where(ar < n0, ssrc, _N)
    dst0 = jnp.where(ar < n0, sdst, 0)
    g1s = jnp.take(ssrc, ar + n0, mode='clip')
    g1d = jnp.take(sdst, ar + n0, mode='clip')
    src1 = jnp.where(ar < n1, g1s, _N)
    dst1 = jnp.where(ar < n1, g1d - _NH, 0)

    def _subshape(a):
        return a.reshape(_EP // _NS, _NS).T.reshape(_NS, _NCH, _EC)

    idx_r = jnp.stack([
        jnp.stack([_subshape(src0), _subshape(dst0)], axis=2),
        jnp.stack([_subshape(src1), _subshape(dst1)], axis=2),
    ])

    def _trips(n):
        per = (n + _NS - 1) // _NS
        return jnp.maximum((per + 2 * _EC - 1) // (2 * _EC), 1)

    trips = jnp.concatenate([_trips(n0).reshape(1), _trips(n1).reshape(1),
                             jnp.zeros((14,), i32)]).astype(i32)

    deg = _deg_kernel(dst_p, val_p).reshape(_NP, 1)

    feat = jnp.concatenate([pos, x, jnp.zeros((_N, 5), f32)], axis=1)
    feat = jnp.pad(feat, ((0, _NP - _N), (0, 0)))
    w_in_p = jnp.concatenate([W_in, jnp.zeros((5, _D), f32)], axis=0)

    h, g, dinv = _tc_init(feat, w_in_p, b_in.reshape(1, _D), W1, deg)

    Ws = (W1, W2, W3, W4)
    bs = (b1.reshape(1, _D), b2.reshape(1, _D),
          b3.reshape(1, _D), b4.reshape(1, _D))
    for t in range(15):
        acc = _msg_kernel(g.reshape(_NP, 2, _H), idx_r,
                          trips).reshape(_NP, _D)
        h, g = _tc_layer(h, acc, g, dinv, bs[t % 4], Ws[(t + 1) % 4])
    acc = _msg_kernel(g.reshape(_NP, 2, _H), idx_r, trips).reshape(_NP, _D)
    out = _tc_head(h, acc, g, dinv, bs[3], W_o1, b_o1.reshape(1, _D),
                   W_o2.reshape(_D, 1))
    return out[:_N] + b_o2


# speedup vs baseline: 1.1824x; 1.1824x over previous
import functools

import jax
import jax.numpy as jnp
from jax import lax
from jax.experimental import pallas as pl
from jax.experimental.pallas import tpu as pltpu
from jax.experimental.pallas import tpu_sc as plsc

_N = 10000
_NP = 10240
_NH = _NP // 2
_E = 160000
_EP = 163840
_D = 256
_H = 128
_NS = 16
_CH = 128
_EC = 64
_NCH = _EP // _NS // _EC
_RPH = _NH // _NS

_mesh = plsc.VectorSubcoreMesh(core_axis_name="c", subcore_axis_name="s")


@functools.partial(
    pl.kernel,
    out_type=jax.ShapeDtypeStruct((_NP // _CH, _CH), jnp.float32),
    mesh=_mesh,
    scratch_types=[
        pltpu.VMEM((_EP // _NS,), jnp.int32),
        pltpu.VMEM((_EP // _NS,), jnp.float32),
        pltpu.VMEM((_NP // _CH, _CH), jnp.float32),
        pltpu.VMEM((_NP // _CH,), jnp.int32),
        pltpu.VMEM_SHARED((_NP // _CH, _CH), jnp.float32),
    ],
    compiler_params=pltpu.CompilerParams(needs_layout_passes=False),
)
def _deg_kernel(dst_hbm, val_hbm, deg_hbm, dstv, valv, part, idv, shdeg):
    c = lax.axis_index("c")
    s = lax.axis_index("s")

    @pl.when(c == 0)
    def _core0():
        nrow = _NP // _CH
        def zrow(i, carry):
            for k in range(_CH // 16):
                part[i, pl.ds(k * 16, 16)] = jnp.zeros((16,), jnp.float32)
            return carry
        lax.fori_loop(0, nrow, zrow, 0)
        pltpu.sync_copy(part.at[pl.ds(0, nrow // _NS)],
                        shdeg.at[pl.ds(s * (nrow // _NS), nrow // _NS)])
        for k in range(nrow // 16):
            idv[pl.ds(k * 16, 16)] = lax.iota(jnp.int32, 16) + (k * 16)
        npe = _EP // _NS
        pltpu.sync_copy(dst_hbm.at[pl.ds(s * npe, npe)], dstv)
        pltpu.sync_copy(val_hbm.at[pl.ds(s * npe, npe)], valv)
        plsc.subcore_barrier()
        def sbody(i, carry):
            d16 = dstv[pl.ds(i * 16, 16)]
            v16 = valv[pl.ds(i * 16, 16)]
            r16 = lax.shift_right_logical(d16, 7)
            c16 = lax.bitwise_and(d16, 127)
            plsc.addupdate_scatter(part, [r16, c16], v16)
            return carry
        lax.fori_loop(0, npe // 16, sbody, 0)
        pltpu.sync_copy(part, shdeg.at[idv], add=True)
        plsc.subcore_barrier()

        @pl.when(s == 0)
        def _writer():
            pltpu.sync_copy(shdeg, part)
            pltpu.sync_copy(part, deg_hbm)


@functools.partial(
    pl.kernel,
    out_type=jax.ShapeDtypeStruct((2, _NH, 2, _H), jnp.float32),
    mesh=_mesh,
    scratch_types=[
        pltpu.VMEM((2, _EC), jnp.int32),
        pltpu.VMEM((2, _EC), jnp.int32),
        pltpu.VMEM((_EC, 2, _H), jnp.float32),
        pltpu.VMEM((_EC, 2, _H), jnp.float32),
        pltpu.VMEM_SHARED((_NH, 2, _H), jnp.float32),
        pltpu.VMEM((16,), jnp.int32),
        pltpu.SemaphoreType.DMA,
        pltpu.SemaphoreType.DMA,
    ],
    compiler_params=pltpu.CompilerParams(needs_layout_passes=False),
)
def _msg_kernel(g_hbm, idx_hbm, trips_hbm, out_hbm, ib0, ib1, rows0, rows1,
                accum, tsm, sem0, sem1):
    c = lax.axis_index("c")
    s = lax.axis_index("s")
    pltpu.sync_copy(trips_hbm, tsm)
    def zrow(i, carry):
        for q in range(2):
            for k in range(_H // 16):
                rows0[i, q, pl.ds(k * 16, 16)] = jnp.zeros((16,), jnp.float32)
        return carry
    lax.fori_loop(0, _EC, zrow, 0)
    base = s * _RPH
    for k in range(_RPH // _EC):
        pltpu.sync_copy(rows0, accum.at[pl.ds(base + k * _EC, _EC)])
    plsc.subcore_barrier()

    tv = tsm[pl.ds(0, 16)]
    ntrip = jnp.sum(jnp.where(lax.iota(jnp.int32, 16) == c, tv, 0))
    nch = ntrip * 2
    it = idx_hbm.at[c, s]
    pltpu.sync_copy(it.at[0], ib0)
    pltpu.async_copy(g_hbm.at[ib0.at[0]], rows0, sem0)
    pltpu.sync_copy(it.at[1], ib1)
    pltpu.async_copy(g_hbm.at[ib1.at[0]], rows1, sem1)
    bufs = ((ib0, rows0, sem0), (ib1, rows1, sem1))

    def step(i, carry):
        jb = i * 2
        for b in range(2):
            j = jb + b
            ib, rows, sem = bufs[b]
            pltpu.make_async_copy(g_hbm.at[ib.at[0]], rows, sem).wait()
            pltpu.sync_copy(rows, accum.at[ib.at[1]], add=True)

            @pl.when(j + 2 < nch)
            def _prefetch():
                pltpu.sync_copy(it.at[j + 2], ib)
                pltpu.async_copy(g_hbm.at[ib.at[0]], rows, sem)
        return carry

    lax.fori_loop(0, ntrip, step, 0)
    plsc.subcore_barrier()
    pltpu.sync_copy(accum.at[pl.ds(base, _RPH)],
                    out_hbm.at[c].at[pl.ds(base, _RPH)])


def _init_body(feat_ref, win_ref, bin_ref, w1_ref, deg_ref,
               h_ref, g_ref, dinv_ref):
    j = pl.program_id(0)
    hn = jnp.dot(feat_ref[...], win_ref[...],
                 preferred_element_type=jnp.float32) + bin_ref[...]
    rid = j * _D + lax.broadcasted_iota(jnp.int32, (_D, 1), 0)
    dinv = jnp.where(rid < _N, lax.rsqrt(deg_ref[...] + 1.0), 0.0)
    g = dinv * jnp.dot(hn, w1_ref[...], preferred_element_type=jnp.float32)
    h_ref[...] = hn
    g_ref[...] = g
    dinv_ref[...] = dinv


def _layer_body(h_ref, a_ref, g_ref, dinv_ref, b_ref, w_ref, ho_ref, go_ref):
    dinv = dinv_ref[...]
    hn = jnp.maximum(h_ref[...] + dinv * (a_ref[...] + g_ref[...])
                     + b_ref[...], 0.0)
    gn = dinv * jnp.dot(hn, w_ref[...], preferred_element_type=jnp.float32)
    ho_ref[...] = hn
    go_ref[...] = gn


def _head_body(h_ref, a_ref, g_ref, dinv_ref, b_ref, wo1_ref, bo1_ref,
               wo2_ref, o_ref):
    dinv = dinv_ref[...]
    hn = jnp.maximum(h_ref[...] + dinv * (a_ref[...] + g_ref[...])
                     + b_ref[...], 0.0)
    t = jnp.dot(hn, wo1_ref[...], preferred_element_type=jnp.float32)
    t = t + bo1_ref[...]
    t = jnp.where(t >= 0, t, 0.01 * t)
    y = jnp.dot(t, wo2_ref[...], preferred_element_type=jnp.float32)
    o_ref[...] = y


_GRID = _NP // _D


def _full(shape):
    return pl.BlockSpec(shape, lambda j: tuple(0 for _ in shape))


_ROW = pl.BlockSpec((_D, _D), lambda j: (j, 0))
_COL = pl.BlockSpec((_D, 1), lambda j: (j, 0))


def _tc_init(feat, w_in, b_in, w1, deg):
    return pl.pallas_call(
        _init_body,
        grid=(_GRID,),
        in_specs=[
            pl.BlockSpec((_D, 8), lambda j: (j, 0)),
            _full((8, _D)),
            _full((1, _D)),
            _full((_D, _D)),
            _COL,
        ],
        out_specs=[_ROW, _ROW, _COL],
        out_shape=[
            jax.ShapeDtypeStruct((_NP, _D), jnp.float32),
            jax.ShapeDtypeStruct((_NP, _D), jnp.float32),
            jax.ShapeDtypeStruct((_NP, 1), jnp.float32),
        ],
    )(feat, w_in, b_in, w1, deg)


def _tc_layer(h, acc, g, dinv, b, w_next):
    return pl.pallas_call(
        _layer_body,
        grid=(_GRID,),
        in_specs=[_ROW, _ROW, _ROW, _COL, _full((1, _D)), _full((_D, _D))],
        out_specs=[_ROW, _ROW],
        out_shape=[
            jax.ShapeDtypeStruct((_NP, _D), jnp.float32),
            jax.ShapeDtypeStruct((_NP, _D), jnp.float32),
        ],
    )(h, acc, g, dinv, b, w_next)


def _tc_head(h, acc, g, dinv, b, w_o1, b_o1, w_o2):
    return pl.pallas_call(
        _head_body,
        grid=(_GRID,),
        in_specs=[_ROW, _ROW, _ROW, _COL, _full((1, _D)), _full((_D, _D)),
                  _full((1, _D)), _full((_D, 1))],
        out_specs=_COL,
        out_shape=jax.ShapeDtypeStruct((_NP, 1), jnp.float32),
    )(h, acc, g, dinv, b, w_o1, b_o1, w_o2)


def kernel(x, pos, edge_index, W_in, b_in, W1, b1, W2, b2, W3, b3, W4, b4,
           W_o1, b_o1, W_o2, b_o2):
    f32 = jnp.float32
    i32 = jnp.int32
    src = edge_index[0]
    dst = edge_index[1]
    pad = _EP - _E

    dst_p = jnp.concatenate([dst, jnp.full((pad,), _N, i32)])
    val_p = jnp.concatenate([jnp.ones((_E,), f32), jnp.zeros((pad,), f32)])

    m1 = dst >= _NH
    n1 = jnp.sum(m1.astype(i32))
    n0 = _E - n1
    order = jnp.argsort(m1, stable=True)
    ssrc = jnp.pad(src[order], (0, _EP - _E))
    sdst = jnp.pad(dst[order], (0, _EP - _E))

    ar = jnp.arange(_EP, dtype=i32)
    src0 = jnp.where(ar < n0, ssrc, _N)
    dst0 = jnp.where(ar < n0, sdst, 0)
    g1s = jnp.take(ssrc, ar + n0, mode='clip')
    g1d = jnp.take(sdst, ar + n0, mode='clip')
    src1 = jnp.where(ar < n1, g1s, _N)
    dst1 = jnp.where(ar < n1, g1d - _NH, 0)

    def _subshape(a):
        return a.reshape(_EP // _NS, _NS).T.reshape(_NS, _NCH, _EC)

    idx_r = jnp.stack([
        jnp.stack([_subshape(src0), _subshape(dst0)], axis=2),
        jnp.stack([_subshape(src1), _subshape(dst1)], axis=2),
    ])

    def _trips(n):
        per = (n + _NS - 1) // _NS
        return jnp.maximum((per + 2 * _EC - 1) // (2 * _EC), 1)

    trips = jnp.concatenate([_trips(n0).reshape(1), _trips(n1).reshape(1),
                             jnp.zeros((14,), i32)]).astype(i32)

    deg = _deg_kernel(dst_p, val_p).reshape(_NP, 1)

    feat = jnp.concatenate([pos, x, jnp.zeros((_N, 5), f32)], axis=1)
    feat = jnp.pad(feat, ((0, _NP - _N), (0, 0)))
    w_in_p = jnp.concatenate([W_in, jnp.zeros((5, _D), f32)], axis=0)

    h, g, dinv = _tc_init(feat, w_in_p, b_in.reshape(1, _D), W1, deg)

    Ws = (W1, W2, W3, W4)
    bs = (b1.reshape(1, _D), b2.reshape(1, _D),
          b3.reshape(1, _D), b4.reshape(1, _D))
    for t in range(15):
        acc = _msg_kernel(g.reshape(_NP, 2, _H), idx_r,
                          trips).reshape(_NP, _D)
        h, g = _tc_layer(h, acc, g, dinv, bs[t % 4], Ws[(t + 1) % 4])
    acc = _msg_kernel(g.reshape(_NP, 2, _H), idx_r, trips).reshape(_NP, _D)
    out = _tc_head(h, acc, g, dinv, bs[3], W_o1, b_o1.reshape(1, _D),
                   W_o2.reshape(_D, 1))
    return out[:_N] + b_o2


# speedup vs baseline: 1.3757x; 1.1635x over previous
import functools

import jax
import jax.numpy as jnp
from jax import lax
from jax.experimental import pallas as pl
from jax.experimental.pallas import tpu as pltpu
from jax.experimental.pallas import tpu_sc as plsc

_N = 10000
_NP = 10240
_NH = _NP // 2
_E = 160000
_EP = 163840
_D = 256
_H = 128
_NS = 16
_CH = 128
_EC = 64
_NCH = _EP // _NS // _EC
_RPH = _NH // _NS

_mesh = plsc.VectorSubcoreMesh(core_axis_name="c", subcore_axis_name="s")


@functools.partial(
    pl.kernel,
    out_type=jax.ShapeDtypeStruct((_NP // _CH, _CH), jnp.float32),
    mesh=_mesh,
    scratch_types=[
        pltpu.VMEM((_EP // _NS,), jnp.int32),
        pltpu.VMEM((_EP // _NS,), jnp.float32),
        pltpu.VMEM((_NP // _CH, _CH), jnp.float32),
        pltpu.VMEM((_NP // _CH,), jnp.int32),
        pltpu.VMEM_SHARED((_NP // _CH, _CH), jnp.float32),
    ],
    compiler_params=pltpu.CompilerParams(needs_layout_passes=False),
)
def _deg_kernel(dst_hbm, val_hbm, deg_hbm, dstv, valv, part, idv, shdeg):
    c = lax.axis_index("c")
    s = lax.axis_index("s")

    @pl.when(c == 0)
    def _core0():
        nrow = _NP // _CH
        def zrow(i, carry):
            for k in range(_CH // 16):
                part[i, pl.ds(k * 16, 16)] = jnp.zeros((16,), jnp.float32)
            return carry
        lax.fori_loop(0, nrow, zrow, 0)
        pltpu.sync_copy(part.at[pl.ds(0, nrow // _NS)],
                        shdeg.at[pl.ds(s * (nrow // _NS), nrow // _NS)])
        for k in range(nrow // 16):
            idv[pl.ds(k * 16, 16)] = lax.iota(jnp.int32, 16) + (k * 16)
        npe = _EP // _NS
        pltpu.sync_copy(dst_hbm.at[pl.ds(s * npe, npe)], dstv)
        pltpu.sync_copy(val_hbm.at[pl.ds(s * npe, npe)], valv)
        plsc.subcore_barrier()
        def sbody(i, carry):
            d16 = dstv[pl.ds(i * 16, 16)]
            v16 = valv[pl.ds(i * 16, 16)]
            r16 = lax.shift_right_logical(d16, 7)
            c16 = lax.bitwise_and(d16, 127)
            plsc.addupdate_scatter(part, [r16, c16], v16)
            return carry
        lax.fori_loop(0, npe // 16, sbody, 0)
        pltpu.sync_copy(part, shdeg.at[idv], add=True)
        plsc.subcore_barrier()

        @pl.when(s == 0)
        def _writer():
            pltpu.sync_copy(shdeg, part)
            pltpu.sync_copy(part, deg_hbm)


@functools.partial(
    pl.kernel,
    out_type=jax.ShapeDtypeStruct((2, _NH, 2, _H), jnp.float32),
    mesh=_mesh,
    scratch_types=[
        pltpu.VMEM((2, _EC), jnp.int32),
        pltpu.VMEM((2, _EC), jnp.int32),
        pltpu.VMEM((_EC, 2, _H), jnp.float32),
        pltpu.VMEM((_EC, 2, _H), jnp.float32),
        pltpu.VMEM_SHARED((_NH, 2, _H), jnp.float32),
        pltpu.VMEM((16,), jnp.int32),
        pltpu.SemaphoreType.DMA,
        pltpu.SemaphoreType.DMA,
    ],
    compiler_params=pltpu.CompilerParams(needs_layout_passes=False),
)
def _msg_kernel(g_hbm, idx_hbm, trips_hbm, out_hbm, ib0, ib1, rows0, rows1,
                accum, tsm, sem0, sem1):
    c = lax.axis_index("c")
    s = lax.axis_index("s")
    pltpu.sync_copy(trips_hbm, tsm)
    def zrow(i, carry):
        for q in range(2):
            for k in range(_H // 16):
                rows0[i, q, pl.ds(k * 16, 16)] = jnp.zeros((16,), jnp.float32)
        return carry
    lax.fori_loop(0, _EC, zrow, 0)
    base = s * _RPH
    for k in range(_RPH // _EC):
        pltpu.sync_copy(rows0, accum.at[pl.ds(base + k * _EC, _EC)])
    plsc.subcore_barrier()

    tv = tsm[pl.ds(0, 16)]
    ntrip = jnp.sum(jnp.where(lax.iota(jnp.int32, 16) == c, tv, 0))
    nch = ntrip * 2
    it = idx_hbm.at[c, s]
    pltpu.sync_copy(it.at[0], ib0)
    pltpu.async_copy(g_hbm.at[ib0.at[0]], rows0, sem0)
    pltpu.sync_copy(it.at[1], ib1)
    pltpu.async_copy(g_hbm.at[ib1.at[0]], rows1, sem1)
    bufs = ((ib0, rows0, sem0), (ib1, rows1, sem1))

    def step(i, carry):
        jb = i * 2
        for b in range(2):
            j = jb + b
            ib, rows, sem = bufs[b]
            pltpu.make_async_copy(g_hbm.at[ib.at[0]], rows, sem).wait()
            pltpu.sync_copy(rows, accum.at[ib.at[1]], add=True)

            @pl.when(j + 2 < nch)
            def _prefetch():
                pltpu.sync_copy(it.at[j + 2], ib)
                pltpu.async_copy(g_hbm.at[ib.at[0]], rows, sem)
        return carry

    lax.fori_loop(0, ntrip, step, 0)
    plsc.subcore_barrier()
    pltpu.sync_copy(accum.at[pl.ds(base, _RPH)],
                    out_hbm.at[c].at[pl.ds(base, _RPH)])


def _init_body(feat_ref, win_ref, bin_ref, w1_ref, deg_ref,
               h_ref, g_ref, dinv_ref):
    j = pl.program_id(0)
    hn = jnp.dot(feat_ref[...], win_ref[...],
                 preferred_element_type=jnp.float32) + bin_ref[...]
    rid = j * _D + lax.broadcasted_iota(jnp.int32, (_D, 1), 0)
    dinv = jnp.where(rid < _N, lax.rsqrt(deg_ref[...] + 1.0), 0.0)
    g = dinv * jnp.dot(hn, w1_ref[...], preferred_element_type=jnp.float32)
    h_ref[...] = hn
    g_ref[...] = g
    dinv_ref[...] = dinv


def _layer_body(h_ref, a_ref, g_ref, dinv_ref, b_ref, w_ref, ho_ref, go_ref):
    dinv = dinv_ref[...]
    hn = jnp.maximum(h_ref[...] + dinv * (a_ref[...] + g_ref[...])
                     + b_ref[...], 0.0)
    gn = dinv * jnp.dot(hn, w_ref[...], preferred_element_type=jnp.float32)
    ho_ref[...] = hn
    go_ref[...] = gn


def _head_body(h_ref, a_ref, g_ref, dinv_ref, b_ref, wo1_ref, bo1_ref,
               wo2_ref, o_ref):
    dinv = dinv_ref[...]
    hn = jnp.maximum(h_ref[...] + dinv * (a_ref[...] + g_ref[...])
                     + b_ref[...], 0.0)
    t = jnp.dot(hn, wo1_ref[...], preferred_element_type=jnp.float32)
    t = t + bo1_ref[...]
    t = jnp.where(t >= 0, t, 0.01 * t)
    y = jnp.dot(t, wo2_ref[...], preferred_element_type=jnp.float32)
    o_ref[...] = y


_GRID = _NP // _D


def _full(shape):
    return pl.BlockSpec(shape, lambda j: tuple(0 for _ in shape))


_ROW = pl.BlockSpec((_D, _D), lambda j: (j, 0))
_COL = pl.BlockSpec((_D, 1), lambda j: (j, 0))


def _tc_init(feat, w_in, b_in, w1, deg):
    return pl.pallas_call(
        _init_body,
        grid=(_GRID,),
        in_specs=[
            pl.BlockSpec((_D, 8), lambda j: (j, 0)),
            _full((8, _D)),
            _full((1, _D)),
            _full((_D, _D)),
            _COL,
        ],
        out_specs=[_ROW, _ROW, _COL],
        out_shape=[
            jax.ShapeDtypeStruct((_NP, _D), jnp.float32),
            jax.ShapeDtypeStruct((_NP, _D), jnp.float32),
            jax.ShapeDtypeStruct((_NP, 1), jnp.float32),
        ],
    )(feat, w_in, b_in, w1, deg)


def _tc_layer(h, acc, g, dinv, b, w_next):
    return pl.pallas_call(
        _layer_body,
        grid=(_GRID,),
        in_specs=[_ROW, _ROW, _ROW, _COL, _full((1, _D)), _full((_D, _D))],
        out_specs=[_ROW, _ROW],
        out_shape=[
            jax.ShapeDtypeStruct((_NP, _D), jnp.float32),
            jax.ShapeDtypeStruct((_NP, _D), jnp.float32),
        ],
    )(h, acc, g, dinv, b, w_next)


def _tc_head(h, acc, g, dinv, b, w_o1, b_o1, w_o2):
    return pl.pallas_call(
        _head_body,
        grid=(_GRID,),
        in_specs=[_ROW, _ROW, _ROW, _COL, _full((1, _D)), _full((_D, _D)),
                  _full((1, _D)), _full((_D, 1))],
        out_specs=_COL,
        out_shape=jax.ShapeDtypeStruct((_NP, 1), jnp.float32),
    )(h, acc, g, dinv, b, w_o1, b_o1, w_o2)


def kernel(x, pos, edge_index, W_in, b_in, W1, b1, W2, b2, W3, b3, W4, b4,
           W_o1, b_o1, W_o2, b_o2):
    f32 = jnp.float32
    i32 = jnp.int32
    src = edge_index[0]
    dst = edge_index[1]
    pad = _EP - _E

    dst_p = jnp.concatenate([dst, jnp.full((pad,), _N, i32)])
    val_p = jnp.concatenate([jnp.ones((_E,), f32), jnp.zeros((pad,), f32)])

    m1 = (dst >= _NH).astype(i32)
    n1 = jnp.sum(m1)
    n0 = _E - n1
    _, osrc, odst = lax.sort((m1, src, dst), num_keys=1, is_stable=True)
    ssrc = jnp.pad(osrc, (0, _EP - _E))
    sdst = jnp.pad(odst, (0, _EP - _E))

    ar = jnp.arange(_EP, dtype=i32)
    src0 = jnp.where(ar < n0, ssrc, _N)
    dst0 = jnp.where(ar < n0, sdst, 0)
    g1s = lax.dynamic_slice(jnp.concatenate([ssrc, ssrc]), (n0,), (_EP,))
    g1d = lax.dynamic_slice(jnp.concatenate([sdst, sdst]), (n0,), (_EP,))
    src1 = jnp.where(ar < n1, g1s, _N)
    dst1 = jnp.where(ar < n1, g1d - _NH, 0)

    def _subshape(a):
        return a.reshape(_EP // _NS, _NS).T.reshape(_NS, _NCH, _EC)

    idx_r = jnp.stack([
        jnp.stack([_subshape(src0), _subshape(dst0)], axis=2),
        jnp.stack([_subshape(src1), _subshape(dst1)], axis=2),
    ])

    def _trips(n):
        per = (n + _NS - 1) // _NS
        return jnp.maximum((per + 2 * _EC - 1) // (2 * _EC), 1)

    trips = jnp.concatenate([_trips(n0).reshape(1), _trips(n1).reshape(1),
                             jnp.zeros((14,), i32)]).astype(i32)

    deg = _deg_kernel(dst_p, val_p).reshape(_NP, 1)

    feat = jnp.concatenate([pos, x, jnp.zeros((_N, 5), f32)], axis=1)
    feat = jnp.pad(feat, ((0, _NP - _N), (0, 0)))
    w_in_p = jnp.concatenate([W_in, jnp.zeros((5, _D), f32)], axis=0)

    h, g, dinv = _tc_init(feat, w_in_p, b_in.reshape(1, _D), W1, deg)

    Ws = (W1, W2, W3, W4)
    bs = (b1.reshape(1, _D), b2.reshape(1, _D),
          b3.reshape(1, _D), b4.reshape(1, _D))
    for t in range(15):
        acc = _msg_kernel(g.reshape(_NP, 2, _H), idx_r,
                          trips).reshape(_NP, _D)
        h, g = _tc_layer(h, acc, g, dinv, bs[t % 4], Ws[(t + 1) % 4])
    acc = _msg_kernel(g.reshape(_NP, 2, _H), idx_r, trips).reshape(_NP, _D)
    out = _tc_head(h, acc, g, dinv, bs[3], W_o1, b_o1.reshape(1, _D),
                   W_o2.reshape(_D, 1))
    return out[:_N] + b_o2
